# baseline (speedup 1.0000x reference)
import functools

import jax
import jax.numpy as jnp
from jax import lax
from jax.experimental import pallas as pl
from jax.experimental.pallas import tpu as pltpu
from jax.experimental.pallas import tpu_sc as plsc

NBUF = 12
AHEAD = 10


@functools.lru_cache(maxsize=None)
def _make_gather(vocab: int, words: int, dim: int, batch: int):
    info = plsc.get_sparse_core_info()
    nc, ns = info.num_cores, info.num_subcores
    nw = nc * ns
    b_per_w = batch // nw
    assert batch == nw * b_per_w

    mesh = plsc.VectorSubcoreMesh(core_axis_name="c", subcore_axis_name="s")

    @functools.partial(
        pl.kernel,
        mesh=mesh,
        out_type=jax.ShapeDtypeStruct((batch, words, dim), jnp.float32),
        scratch_types=[
            pltpu.VMEM((b_per_w, words), jnp.int32),
            pltpu.VMEM((NBUF, words, dim), jnp.float32),
            pltpu.SemaphoreType.DMA,
            pltpu.SemaphoreType.DMA,
            pltpu.SemaphoreType.DMA,
        ],
        compiler_params=pltpu.CompilerParams(use_tc_tiling_on_sc=False),
    )
    def gather_kernel(x_hbm, table_hbm, out_hbm, idx_v, bufs, isem, gsem, ssem):
        wid = lax.axis_index("s") * nc + lax.axis_index("c")
        i0 = wid * b_per_w

        pltpu.async_copy(x_hbm.at[pl.ds(i0, b_per_w)], idx_v, isem).wait()

        def fire_gather(j):
            m = lax.rem(j, NBUF)
            pltpu.async_copy(table_hbm.at[idx_v.at[j]], bufs.at[m], gsem)

        def drain_gather(j):
            m = lax.rem(j, NBUF)
            pltpu.make_async_copy(
                table_hbm.at[pl.ds(0, words)], bufs.at[m], gsem
            ).wait()

        def fire_store(j):
            m = lax.rem(j, NBUF)
            pltpu.make_async_copy(bufs.at[m], out_hbm.at[i0 + j], ssem).start()

        def drain_store(j):
            m = lax.rem(j, NBUF)
            pltpu.make_async_copy(bufs.at[m], out_hbm.at[i0], ssem).wait()

        for j in range(AHEAD):
            fire_gather(j)

        def step(j, carry):
            drain_gather(j)

            @pl.when(j + AHEAD < b_per_w)
            def _():
                @pl.when(j + AHEAD >= NBUF)
                def _():
                    drain_store(j + AHEAD - NBUF)

                fire_gather(j + AHEAD)

            fire_store(j)
            return carry

        lax.fori_loop(0, b_per_w, step, 0)
        for j in range(b_per_w - NBUF, b_per_w):
            drain_store(j)

    return gather_kernel, nw


def kernel(x, table):
    vocab, dim = table.shape
    batch, words = x.shape
    gather_kernel, nw = _make_gather(vocab, words, dim, batch)
    return gather_kernel(x.astype(jnp.int32), table)
